# baseline (speedup 1.0000x reference)
import functools
import math

import jax
import jax.numpy as jnp
from jax import lax
from jax.experimental import pallas as pl
from jax.experimental.pallas import tpu as pltpu
from jax.experimental.pallas import tpu_sc as plsc

RC = 3.0
B = 1.0
A = 1.0
N_ATOMS = 100000
N_EDGES = 6400000

NC = 2
NS = 16
L = 16
NW = NC * NS
EPW = N_EDGES // NW
CHUNK = 800
NCH = EPW // CHUNK
IPC = CHUNK // L
U = 5
AR = 896
AC = 128
RPT = AR // NS


@functools.partial(
    pl.kernel,
    out_type=jax.ShapeDtypeStruct((NC, AR, AC), jnp.float32),
    mesh=plsc.VectorSubcoreMesh(
        core_axis_name="c", subcore_axis_name="s", num_cores=NC,
        num_subcores=NS,
    ),
    scratch_types=[
        pltpu.VMEM((AR, AC), jnp.float32),
        pltpu.VMEM((CHUNK,), jnp.float32),
        pltpu.VMEM((CHUNK,), jnp.float32),
        pltpu.VMEM((2 * CHUNK,), jnp.int32),
        pltpu.VMEM((2 * CHUNK,), jnp.int32),
        pltpu.VMEM((AR,), jnp.int32),
        pltpu.VMEM_SHARED((AR, AC), jnp.float32),
        pltpu.SemaphoreType.DMA,
        pltpu.SemaphoreType.DMA,
        pltpu.SemaphoreType.DMA,
        pltpu.SemaphoreType.DMA,
    ],
    compiler_params=pltpu.CompilerParams(needs_layout_passes=False),
)
def _sc_segsum(dist_hbm, ind_hbm, out_hbm, acc, dbuf0, dbuf1, ibuf0, ibuf1,
               rowidx, stage, sd0, sd1, si0, si1):
    cid = lax.axis_index("c")
    sid = lax.axis_index("s")
    wid = sid * NC + cid
    ebase = wid * EPW
    io = lax.iota(jnp.int32, L)
    zero = jnp.zeros((L,), jnp.float32)
    half_a = jnp.float32(0.5 * A)
    half_e0 = jnp.float32(0.5 * A * math.exp(-RC / B))

    dbufs = (dbuf0, dbuf1)
    ibufs = (ibuf0, ibuf1)
    dsems = (sd0, sd1)
    isems = (si0, si1)

    def issue(ch, buf):
        eb = ebase + ch * CHUNK
        pltpu.async_copy(dist_hbm.at[pl.ds(eb, CHUNK)], dbufs[buf], dsems[buf])
        pltpu.async_copy(ind_hbm.at[pl.ds(2 * eb, 2 * CHUNK)], ibufs[buf],
                         isems[buf])

    def wait(buf):
        pltpu.make_async_copy(dist_hbm.at[pl.ds(0, CHUNK)], dbufs[buf],
                              dsems[buf]).wait()
        pltpu.make_async_copy(ind_hbm.at[pl.ds(0, 2 * CHUNK)], ibufs[buf],
                              isems[buf]).wait()

    def process(buf):
        def group(g, carry):
            for u in range(U):
                b16 = (g * U + u) * L
                d = dbufs[buf][pl.ds(b16, L)]
                idxv = plsc.load_gather(ibufs[buf], [io * 2 + 2 * b16])
                env = half_a * jnp.exp(-d) - half_e0
                row = lax.shift_right_logical(idxv, 7)
                col = lax.bitwise_and(idxv, AC - 1)
                plsc.addupdate_scatter(acc, [row, col], env)
            return carry
        lax.fori_loop(0, IPC // U, group, 0)

    issue(0, 0)
    issue(1, 1)

    def zinit(j, carry):
        for k in range(AC // L):
            acc[j, pl.ds(k * L, L)] = zero
        return carry
    lax.fori_loop(0, AR, zinit, 0)

    def iinit(j, carry):
        rowidx[pl.ds(j * L, L)] = io + j * L
        return carry
    lax.fori_loop(0, AR // L, iinit, 0)

    pltpu.sync_copy(acc.at[pl.ds(0, RPT)], stage.at[pl.ds(sid * RPT, RPT)])

    def outer(j, carry):
        wait(0)
        process(0)
        issue(2 * j + 2, 0)
        wait(1)
        process(1)
        issue(2 * j + 3, 1)
        return carry
    lax.fori_loop(0, NCH // 2 - 1, outer, 0)
    wait(0)
    process(0)
    wait(1)
    process(1)

    plsc.subcore_barrier()
    for r in range(NS):
        @pl.when(sid == r)
        def _():
            pltpu.sync_copy(acc, stage.at[rowidx], add=True)
        plsc.subcore_barrier()
    pltpu.sync_copy(stage.at[pl.ds(sid * RPT, RPT)],
                    out_hbm.at[cid, pl.ds(sid * RPT, RPT)])


def _tc_add(a_ref, b_ref, o_ref):
    o_ref[...] = a_ref[...] + b_ref[...]


_combine = pl.pallas_call(
    _tc_add,
    out_shape=jax.ShapeDtypeStruct((AR, AC), jnp.float32),
)


def kernel(dist, ind_1, ind_2):
    del ind_1
    ind_flat = ind_2.astype(jnp.int32).reshape(-1)
    partials = _sc_segsum(dist, ind_flat)
    out = _combine(partials[0], partials[1])
    return out.reshape(-1)[:N_ATOMS]


# speedup vs baseline: 2.8753x; 2.8753x over previous
import functools
import math

import jax
import jax.numpy as jnp
from jax import lax
from jax.experimental import pallas as pl
from jax.experimental.pallas import tpu as pltpu
from jax.experimental.pallas import tpu_sc as plsc

RC = 3.0
B = 1.0
A = 1.0
N_ATOMS = 100000
N_EDGES = 6400000

NC = 2
NS = 16
L = 16
NW = NC * NS
EPW = N_EDGES // NW
CHUNK = 800
NCH = EPW // CHUNK
IPC = CHUNK // L
U = 5
AR = 896
AC = 128
RPT = AR // NS


XC = 400
XNCH = EPW // XC
XIP = XC // L
XU = 5


@functools.partial(
    pl.kernel,
    out_type=jax.ShapeDtypeStruct((N_EDGES,), jnp.int32),
    mesh=plsc.VectorSubcoreMesh(
        core_axis_name="c", subcore_axis_name="s", num_cores=NC,
        num_subcores=NS,
    ),
    scratch_types=[
        pltpu.VMEM((XC, 2), jnp.int32),
        pltpu.VMEM((XC, 2), jnp.int32),
        pltpu.VMEM((XC,), jnp.int32),
        pltpu.VMEM((XC,), jnp.int32),
        pltpu.SemaphoreType.DMA,
        pltpu.SemaphoreType.DMA,
        pltpu.SemaphoreType.DMA,
        pltpu.SemaphoreType.DMA,
    ],
    compiler_params=pltpu.CompilerParams(needs_layout_passes=False),
)
def _sc_extract(ind_hbm, ids_hbm, pbuf0, pbuf1, obuf0, obuf1,
                si0, si1, so0, so1):
    cid = lax.axis_index("c")
    sid = lax.axis_index("s")
    wid = sid * NC + cid
    ebase = wid * EPW
    io = lax.iota(jnp.int32, L)
    zero16 = jnp.zeros((L,), jnp.int32)

    pbufs = (pbuf0, pbuf1)
    obufs = (obuf0, obuf1)
    isems = (si0, si1)
    osems = (so0, so1)

    def issue(ch, buf):
        eb = ebase + ch * XC
        pltpu.async_copy(ind_hbm.at[pl.ds(eb, XC), :], pbufs[buf],
                         isems[buf])

    def wait_in(buf):
        pltpu.make_async_copy(ind_hbm.at[pl.ds(0, XC), :], pbufs[buf],
                              isems[buf]).wait()

    def flush(ch, buf):
        eb = ebase + ch * XC
        pltpu.async_copy(obufs[buf], ids_hbm.at[pl.ds(eb, XC)], osems[buf])

    def wait_out(buf):
        pltpu.make_async_copy(obufs[buf], ids_hbm.at[pl.ds(0, XC)],
                              osems[buf]).wait()

    def extract(buf):
        def group(g, carry):
            for u in range(XU):
                j = g * XU + u
                idv = plsc.load_gather(pbufs[buf], [io + j * L, zero16])
                obufs[buf][pl.ds(j * L, L)] = idv
            return carry
        lax.fori_loop(0, XIP // XU, group, 0)

    issue(0, 0)
    issue(1, 1)

    def outer(j, carry):
        wait_in(0)
        @pl.when(j > 0)
        def _():
            wait_out(0)
        extract(0)
        flush(2 * j, 0)
        @pl.when(j < XNCH // 2 - 1)
        def _():
            issue(2 * j + 2, 0)

        wait_in(1)
        @pl.when(j > 0)
        def _():
            wait_out(1)
        extract(1)
        flush(2 * j + 1, 1)
        @pl.when(j < XNCH // 2 - 1)
        def _():
            issue(2 * j + 3, 1)
        return carry
    lax.fori_loop(0, XNCH // 2, outer, 0)
    wait_out(0)
    wait_out(1)


@functools.partial(
    pl.kernel,
    out_type=jax.ShapeDtypeStruct((NC, AR, AC), jnp.float32),
    mesh=plsc.VectorSubcoreMesh(
        core_axis_name="c", subcore_axis_name="s", num_cores=NC,
        num_subcores=NS,
    ),
    scratch_types=[
        pltpu.VMEM((AR, AC), jnp.float32),
        pltpu.VMEM((CHUNK,), jnp.float32),
        pltpu.VMEM((CHUNK,), jnp.float32),
        pltpu.VMEM((CHUNK,), jnp.int32),
        pltpu.VMEM((CHUNK,), jnp.int32),
        pltpu.VMEM((AR,), jnp.int32),
        pltpu.VMEM_SHARED((AR, AC), jnp.float32),
        pltpu.SemaphoreType.DMA,
        pltpu.SemaphoreType.DMA,
        pltpu.SemaphoreType.DMA,
        pltpu.SemaphoreType.DMA,
    ],
    compiler_params=pltpu.CompilerParams(needs_layout_passes=False),
)
def _sc_segsum(dist_hbm, ind_hbm, out_hbm, acc, dbuf0, dbuf1, ibuf0, ibuf1,
               rowidx, stage, sd0, sd1, si0, si1):
    cid = lax.axis_index("c")
    sid = lax.axis_index("s")
    wid = sid * NC + cid
    ebase = wid * EPW
    io = lax.iota(jnp.int32, L)
    zero = jnp.zeros((L,), jnp.float32)
    half_a = jnp.float32(0.5 * A)
    half_e0 = jnp.float32(0.5 * A * math.exp(-RC / B))

    dbufs = (dbuf0, dbuf1)
    ibufs = (ibuf0, ibuf1)
    dsems = (sd0, sd1)
    isems = (si0, si1)

    def issue(ch, buf):
        eb = ebase + ch * CHUNK
        pltpu.async_copy(dist_hbm.at[pl.ds(eb, CHUNK)], dbufs[buf], dsems[buf])
        pltpu.async_copy(ind_hbm.at[pl.ds(eb, CHUNK)], ibufs[buf],
                         isems[buf])

    def wait(buf):
        pltpu.make_async_copy(dist_hbm.at[pl.ds(0, CHUNK)], dbufs[buf],
                              dsems[buf]).wait()
        pltpu.make_async_copy(ind_hbm.at[pl.ds(0, CHUNK)], ibufs[buf],
                              isems[buf]).wait()

    def process(buf):
        def group(g, carry):
            for u in range(U):
                b16 = (g * U + u) * L
                d = dbufs[buf][pl.ds(b16, L)]
                idxv = ibufs[buf][pl.ds(b16, L)]
                env = half_a * jnp.exp(-d) - half_e0
                row = lax.shift_right_logical(idxv, 7)
                col = lax.bitwise_and(idxv, AC - 1)
                plsc.addupdate_scatter(acc, [row, col], env)
            return carry
        lax.fori_loop(0, IPC // U, group, 0)

    issue(0, 0)
    issue(1, 1)

    def zinit(j, carry):
        for k in range(AC // L):
            acc[j, pl.ds(k * L, L)] = zero
        return carry
    lax.fori_loop(0, AR, zinit, 0)

    def iinit(j, carry):
        rowidx[pl.ds(j * L, L)] = io + j * L
        return carry
    lax.fori_loop(0, AR // L, iinit, 0)

    pltpu.sync_copy(acc.at[pl.ds(0, RPT)], stage.at[pl.ds(sid * RPT, RPT)])

    def outer(j, carry):
        wait(0)
        process(0)
        issue(2 * j + 2, 0)
        wait(1)
        process(1)
        issue(2 * j + 3, 1)
        return carry
    lax.fori_loop(0, NCH // 2 - 1, outer, 0)
    wait(0)
    process(0)
    wait(1)
    process(1)

    plsc.subcore_barrier()
    for r in range(NS):
        @pl.when(sid == r)
        def _():
            pltpu.sync_copy(acc, stage.at[rowidx], add=True)
        plsc.subcore_barrier()
    pltpu.sync_copy(stage.at[pl.ds(sid * RPT, RPT)],
                    out_hbm.at[cid, pl.ds(sid * RPT, RPT)])


def _tc_add(a_ref, b_ref, o_ref):
    o_ref[...] = a_ref[...] + b_ref[...]


_combine = pl.pallas_call(
    _tc_add,
    out_shape=jax.ShapeDtypeStruct((AR, AC), jnp.float32),
)


def kernel(dist, ind_1, ind_2):
    del ind_1
    ids = _sc_extract(ind_2.astype(jnp.int32))
    partials = _sc_segsum(dist, ids)
    out = _combine(partials[0], partials[1])
    return out.reshape(-1)[:N_ATOMS]


# speedup vs baseline: 30.3437x; 10.5533x over previous
import functools
import math

import jax
import jax.numpy as jnp
from jax import lax
from jax.experimental import pallas as pl
from jax.experimental.pallas import tpu as pltpu
from jax.experimental.pallas import tpu_sc as plsc

RC = 3.0
B = 1.0
A = 1.0
N_ATOMS = 100000
N_EDGES = 6400000

NC = 2
NS = 16
L = 16
NW = NC * NS
CHUNK = 1024
TOTCH = N_EDGES // CHUNK
IPC = CHUNK // L
U = 8
AR = 896
AC = 128
RPT = AR // NS


@functools.partial(
    pl.kernel,
    out_type=jax.ShapeDtypeStruct((NC, AR, AC), jnp.float32),
    mesh=plsc.VectorSubcoreMesh(
        core_axis_name="c", subcore_axis_name="s", num_cores=NC,
        num_subcores=NS,
    ),
    scratch_types=[
        pltpu.VMEM((AR, AC), jnp.float32),
        pltpu.VMEM((CHUNK,), jnp.float32),
        pltpu.VMEM((CHUNK,), jnp.float32),
        pltpu.VMEM((CHUNK,), jnp.int32),
        pltpu.VMEM((CHUNK,), jnp.int32),
        pltpu.VMEM((AR,), jnp.int32),
        pltpu.VMEM_SHARED((AR, AC), jnp.float32),
        pltpu.SemaphoreType.DMA,
        pltpu.SemaphoreType.DMA,
        pltpu.SemaphoreType.DMA,
        pltpu.SemaphoreType.DMA,
    ],
    compiler_params=pltpu.CompilerParams(needs_layout_passes=False),
)
def _sc_segsum(dist_hbm, ind_hbm, out_hbm, acc, dbuf0, dbuf1, ibuf0, ibuf1,
               rowidx, stage, sd0, sd1, si0, si1):
    cid = lax.axis_index("c")
    sid = lax.axis_index("s")
    wid = sid * NC + cid
    n_w = TOTCH // NW + jnp.where(wid < TOTCH % NW, 1, 0)
    io = lax.iota(jnp.int32, L)
    zero = jnp.zeros((L,), jnp.float32)
    half_a = jnp.float32(0.5 * A)
    half_e0 = jnp.float32(0.5 * A * math.exp(-RC / B))

    dbufs = (dbuf0, dbuf1)
    ibufs = (ibuf0, ibuf1)
    dsems = (sd0, sd1)
    isems = (si0, si1)

    def issue(k, buf):
        eb = (wid + NW * k) * CHUNK
        pltpu.async_copy(dist_hbm.at[pl.ds(eb, CHUNK)], dbufs[buf], dsems[buf])
        pltpu.async_copy(ind_hbm.at[0, pl.ds(eb, CHUNK)], ibufs[buf],
                         isems[buf])

    def wait(buf):
        pltpu.make_async_copy(dist_hbm.at[pl.ds(0, CHUNK)], dbufs[buf],
                              dsems[buf]).wait()
        pltpu.make_async_copy(ind_hbm.at[0, pl.ds(0, CHUNK)], ibufs[buf],
                              isems[buf]).wait()

    def process(buf):
        def group(g, carry):
            for u in range(U):
                b16 = (g * U + u) * L
                d = dbufs[buf][pl.ds(b16, L)]
                idxv = ibufs[buf][pl.ds(b16, L)]
                env = half_a * jnp.exp(-d) - half_e0
                row = lax.shift_right_logical(idxv, 7)
                col = lax.bitwise_and(idxv, AC - 1)
                plsc.addupdate_scatter(acc, [row, col], env)
            return carry
        lax.fori_loop(0, IPC // U, group, 0)

    @pl.when(n_w > 0)
    def _():
        issue(0, 0)

    @pl.when(n_w > 1)
    def _():
        issue(1, 1)

    def zinit(j, carry):
        for k in range(AC // L):
            acc[j, pl.ds(k * L, L)] = zero
        return carry
    lax.fori_loop(0, AR, zinit, 0)

    def iinit(j, carry):
        rowidx[pl.ds(j * L, L)] = io + j * L
        return carry
    lax.fori_loop(0, AR // L, iinit, 0)

    pltpu.sync_copy(acc.at[pl.ds(0, RPT)], stage.at[pl.ds(sid * RPT, RPT)])

    def outer(j, carry):
        k0 = 2 * j
        k1 = 2 * j + 1

        @pl.when(k0 < n_w)
        def _():
            wait(0)
            process(0)

        @pl.when(k0 + 2 < n_w)
        def _():
            issue(k0 + 2, 0)

        @pl.when(k1 < n_w)
        def _():
            wait(1)
            process(1)

        @pl.when(k1 + 2 < n_w)
        def _():
            issue(k1 + 2, 1)

        return carry
    lax.fori_loop(0, (n_w + 1) // 2, outer, 0)

    plsc.subcore_barrier()
    for r in range(NS):
        @pl.when(sid == r)
        def _():
            pltpu.sync_copy(acc, stage.at[rowidx], add=True)
        plsc.subcore_barrier()
    pltpu.sync_copy(stage.at[pl.ds(sid * RPT, RPT)],
                    out_hbm.at[cid, pl.ds(sid * RPT, RPT)])


def _tc_add(a_ref, b_ref, o_ref):
    o_ref[...] = a_ref[...] + b_ref[...]


_combine = pl.pallas_call(
    _tc_add,
    out_shape=jax.ShapeDtypeStruct((AR, AC), jnp.float32),
)


def kernel(dist, ind_1, ind_2):
    del ind_1
    idt = ind_2.astype(jnp.int32).T
    partials = _sc_segsum(dist, idt)
    out = _combine(partials[0], partials[1])
    return out.reshape(-1)[:N_ATOMS]


# speedup vs baseline: 48.2562x; 1.5903x over previous
import functools
import math

import jax
import jax.numpy as jnp
from jax import lax
from jax.experimental import pallas as pl
from jax.experimental.pallas import tpu as pltpu
from jax.experimental.pallas import tpu_sc as plsc

RC = 3.0
B = 1.0
A = 1.0
N_ATOMS = 100000
N_EDGES = 6400000

NC = 2
NS = 16
L = 16
NW = NC * NS
CHUNK = 1024
TOTCH = N_EDGES // CHUNK
IPC = CHUNK // L
U = 8
AR = 896
AC = 128
RPT = AR // NS


@functools.partial(
    pl.kernel,
    out_type=jax.ShapeDtypeStruct((NC, AR, AC), jnp.float32),
    mesh=plsc.VectorSubcoreMesh(
        core_axis_name="c", subcore_axis_name="s", num_cores=NC,
        num_subcores=NS,
    ),
    scratch_types=[
        pltpu.VMEM((AR, AC), jnp.float32),
        pltpu.VMEM((CHUNK,), jnp.float32),
        pltpu.VMEM((CHUNK,), jnp.float32),
        pltpu.VMEM((CHUNK,), jnp.int32),
        pltpu.VMEM((CHUNK,), jnp.int32),
        pltpu.VMEM((AR,), jnp.int32),
        pltpu.VMEM_SHARED((AR, AC), jnp.float32),
        pltpu.SemaphoreType.DMA,
        pltpu.SemaphoreType.DMA,
        pltpu.SemaphoreType.DMA,
        pltpu.SemaphoreType.DMA,
    ],
    compiler_params=pltpu.CompilerParams(needs_layout_passes=False),
)
def _sc_segsum(dist_hbm, ind_hbm, out_hbm, acc, dbuf0, dbuf1, ibuf0, ibuf1,
               rowidx, stage, sd0, sd1, si0, si1):
    cid = lax.axis_index("c")
    sid = lax.axis_index("s")
    wid = sid * NC + cid
    n_w = TOTCH // NW + jnp.where(wid < TOTCH % NW, 1, 0)
    io = lax.iota(jnp.int32, L)
    zero = jnp.zeros((L,), jnp.float32)
    half_a = jnp.float32(0.5 * A)
    half_e0 = jnp.float32(0.5 * A * math.exp(-RC / B))

    dbufs = (dbuf0, dbuf1)
    ibufs = (ibuf0, ibuf1)
    dsems = (sd0, sd1)
    isems = (si0, si1)

    def issue(k, buf):
        eb = (wid + NW * k) * CHUNK
        pltpu.async_copy(dist_hbm.at[pl.ds(eb, CHUNK)], dbufs[buf], dsems[buf])
        pltpu.async_copy(ind_hbm.at[0, pl.ds(eb, CHUNK)], ibufs[buf],
                         isems[buf])

    def wait(buf):
        pltpu.make_async_copy(dist_hbm.at[pl.ds(0, CHUNK)], dbufs[buf],
                              dsems[buf]).wait()
        pltpu.make_async_copy(ind_hbm.at[0, pl.ds(0, CHUNK)], ibufs[buf],
                              isems[buf]).wait()

    def process(buf):
        @plsc.parallel_loop(0, IPC, 1, unroll=U)
        def body(i):
            b16 = i * L
            d = dbufs[buf][pl.ds(b16, L)]
            idxv = ibufs[buf][pl.ds(b16, L)]
            env = half_a * jnp.exp(-d) - half_e0
            row = lax.shift_right_logical(idxv, 7)
            col = lax.bitwise_and(idxv, AC - 1)
            plsc.addupdate_scatter(acc, [row, col], env)

    @pl.when(n_w > 0)
    def _():
        issue(0, 0)

    @pl.when(n_w > 1)
    def _():
        issue(1, 1)

    def zinit(j, carry):
        for k in range(AC // L):
            acc[j, pl.ds(k * L, L)] = zero
        return carry
    lax.fori_loop(0, AR, zinit, 0)

    def iinit(j, carry):
        rowidx[pl.ds(j * L, L)] = io + j * L
        return carry
    lax.fori_loop(0, AR // L, iinit, 0)

    pltpu.sync_copy(acc.at[pl.ds(0, RPT)], stage.at[pl.ds(sid * RPT, RPT)])

    def outer(j, carry):
        k0 = 2 * j
        k1 = 2 * j + 1

        @pl.when(k0 < n_w)
        def _():
            wait(0)
            process(0)

        @pl.when(k0 + 2 < n_w)
        def _():
            issue(k0 + 2, 0)

        @pl.when(k1 < n_w)
        def _():
            wait(1)
            process(1)

        @pl.when(k1 + 2 < n_w)
        def _():
            issue(k1 + 2, 1)

        return carry
    lax.fori_loop(0, (n_w + 1) // 2, outer, 0)

    plsc.subcore_barrier()
    for r in range(NS):
        @pl.when(sid == r)
        def _():
            pltpu.sync_copy(acc, stage.at[rowidx], add=True)
        plsc.subcore_barrier()
    pltpu.sync_copy(stage.at[pl.ds(sid * RPT, RPT)],
                    out_hbm.at[cid, pl.ds(sid * RPT, RPT)])


def _tc_add(a_ref, b_ref, o_ref):
    o_ref[...] = a_ref[...] + b_ref[...]


_combine = pl.pallas_call(
    _tc_add,
    out_shape=jax.ShapeDtypeStruct((AR, AC), jnp.float32),
)


def kernel(dist, ind_1, ind_2):
    del ind_1
    idt = ind_2.astype(jnp.int32).T
    partials = _sc_segsum(dist, idt)
    out = _combine(partials[0], partials[1])
    return out.reshape(-1)[:N_ATOMS]


# speedup vs baseline: 74.7091x; 1.5482x over previous
import functools
import math

import jax
import jax.numpy as jnp
from jax import lax
from jax.experimental import pallas as pl
from jax.experimental.pallas import tpu as pltpu
from jax.experimental.pallas import tpu_sc as plsc

RC = 3.0
B = 1.0
A = 1.0
N_ATOMS = 100000
N_EDGES = 6400000

NC = 2
NS = 16
L = 16
NW = NC * NS
CHUNK = 1024
TOTCH = N_EDGES // CHUNK
IPC = CHUNK // L
U = 8
AR = 896
AC = 128
RPT = AR // NS


@functools.partial(
    pl.kernel,
    out_type=jax.ShapeDtypeStruct((NC, AR, AC), jnp.float32),
    mesh=plsc.VectorSubcoreMesh(
        core_axis_name="c", subcore_axis_name="s", num_cores=NC,
        num_subcores=NS,
    ),
    scratch_types=[
        pltpu.VMEM((AR, AC), jnp.float32),
        pltpu.VMEM((CHUNK,), jnp.float32),
        pltpu.VMEM((CHUNK,), jnp.float32),
        pltpu.VMEM((CHUNK,), jnp.int32),
        pltpu.VMEM((CHUNK,), jnp.int32),
        pltpu.VMEM((AR,), jnp.int32),
        pltpu.VMEM_SHARED((AR, AC), jnp.float32),
        pltpu.SemaphoreType.DMA,
        pltpu.SemaphoreType.DMA,
        pltpu.SemaphoreType.DMA,
        pltpu.SemaphoreType.DMA,
    ],
    compiler_params=pltpu.CompilerParams(needs_layout_passes=False),
)
def _sc_segsum(dist_hbm, ind_hbm, out_hbm, acc, dbuf0, dbuf1, ibuf0, ibuf1,
               rowidx, stage, sd0, sd1, si0, si1):
    cid = lax.axis_index("c")
    sid = lax.axis_index("s")
    wid = sid * NC + cid
    n_w = TOTCH // NW + jnp.where(wid < TOTCH % NW, 1, 0)
    io = lax.iota(jnp.int32, L)
    zero = jnp.zeros((L,), jnp.float32)
    half_a = jnp.float32(0.5 * A)
    half_e0 = jnp.float32(0.5 * A * math.exp(-RC / B))

    dbufs = (dbuf0, dbuf1)
    ibufs = (ibuf0, ibuf1)
    dsems = (sd0, sd1)
    isems = (si0, si1)

    def issue(k, buf):
        eb = (wid + NW * k) * CHUNK
        pltpu.async_copy(dist_hbm.at[pl.ds(eb, CHUNK)], dbufs[buf], dsems[buf])
        pltpu.async_copy(ind_hbm.at[0, pl.ds(eb, CHUNK)], ibufs[buf],
                         isems[buf])

    def wait(buf):
        pltpu.make_async_copy(dist_hbm.at[pl.ds(0, CHUNK)], dbufs[buf],
                              dsems[buf]).wait()
        pltpu.make_async_copy(ind_hbm.at[0, pl.ds(0, CHUNK)], ibufs[buf],
                              isems[buf]).wait()

    def process(buf):
        @plsc.parallel_loop(0, IPC, 1, unroll=U)
        def body(i):
            b16 = i * L
            d = dbufs[buf][pl.ds(b16, L)]
            idxv = ibufs[buf][pl.ds(b16, L)]
            env = half_a * jnp.exp(-d) - half_e0
            row = lax.shift_right_logical(idxv, 7)
            col = lax.bitwise_and(idxv, AC - 1)
            plsc.addupdate_scatter(acc, [row, col], env)

    @pl.when(n_w > 0)
    def _():
        issue(0, 0)

    @pl.when(n_w > 1)
    def _():
        issue(1, 1)

    def zinit(j, carry):
        for k in range(AC // L):
            acc[j, pl.ds(k * L, L)] = zero
        return carry
    lax.fori_loop(0, AR, zinit, 0)

    def iinit(j, carry):
        rowidx[pl.ds(j * L, L)] = io + j * L
        return carry
    lax.fori_loop(0, AR // L, iinit, 0)

    pltpu.sync_copy(acc.at[pl.ds(0, RPT)], stage.at[pl.ds(sid * RPT, RPT)])

    def outer(j, carry):
        k0 = 2 * j
        k1 = 2 * j + 1

        @pl.when(k0 < n_w)
        def _():
            wait(0)
            process(0)

        @pl.when(k0 + 2 < n_w)
        def _():
            issue(k0 + 2, 0)

        @pl.when(k1 < n_w)
        def _():
            wait(1)
            process(1)

        @pl.when(k1 + 2 < n_w)
        def _():
            issue(k1 + 2, 1)

        return carry
    lax.fori_loop(0, (n_w + 1) // 2, outer, 0)

    plsc.subcore_barrier()
    pltpu.sync_copy(acc, stage.at[rowidx], add=True)
    plsc.subcore_barrier()
    pltpu.sync_copy(stage.at[pl.ds(sid * RPT, RPT)],
                    out_hbm.at[cid, pl.ds(sid * RPT, RPT)])


def _tc_add(a_ref, b_ref, o_ref):
    o_ref[...] = a_ref[...] + b_ref[...]


_combine = pl.pallas_call(
    _tc_add,
    out_shape=jax.ShapeDtypeStruct((AR, AC), jnp.float32),
)


def kernel(dist, ind_1, ind_2):
    del ind_1
    idt = ind_2.astype(jnp.int32).T
    partials = _sc_segsum(dist, idt)
    out = _combine(partials[0], partials[1])
    return out.reshape(-1)[:N_ATOMS]


# speedup vs baseline: 74.9196x; 1.0028x over previous
import functools
import math

import jax
import jax.numpy as jnp
from jax import lax
from jax.experimental import pallas as pl
from jax.experimental.pallas import tpu as pltpu
from jax.experimental.pallas import tpu_sc as plsc

RC = 3.0
B = 1.0
A = 1.0
N_ATOMS = 100000
N_EDGES = 6400000

NC = 2
NS = 16
L = 16
NW = NC * NS
CHUNK = 1024
TOTCH = N_EDGES // CHUNK
IPC = CHUNK // L
U = 16
AR = 896
AC = 128
RPT = AR // NS


@functools.partial(
    pl.kernel,
    out_type=jax.ShapeDtypeStruct((NC, AR, AC), jnp.float32),
    mesh=plsc.VectorSubcoreMesh(
        core_axis_name="c", subcore_axis_name="s", num_cores=NC,
        num_subcores=NS,
    ),
    scratch_types=[
        pltpu.VMEM((AR, AC), jnp.float32),
        pltpu.VMEM((CHUNK,), jnp.float32),
        pltpu.VMEM((CHUNK,), jnp.float32),
        pltpu.VMEM((CHUNK,), jnp.int32),
        pltpu.VMEM((CHUNK,), jnp.int32),
        pltpu.VMEM((AR,), jnp.int32),
        pltpu.VMEM_SHARED((AR, AC), jnp.float32),
        pltpu.SemaphoreType.DMA,
        pltpu.SemaphoreType.DMA,
        pltpu.SemaphoreType.DMA,
        pltpu.SemaphoreType.DMA,
    ],
    compiler_params=pltpu.CompilerParams(needs_layout_passes=False),
)
def _sc_segsum(dist_hbm, ind_hbm, out_hbm, acc, dbuf0, dbuf1, ibuf0, ibuf1,
               rowidx, stage, sd0, sd1, si0, si1):
    cid = lax.axis_index("c")
    sid = lax.axis_index("s")
    wid = sid * NC + cid
    n_w = TOTCH // NW + jnp.where(wid < TOTCH % NW, 1, 0)
    io = lax.iota(jnp.int32, L)
    zero = jnp.zeros((L,), jnp.float32)
    half_a = jnp.float32(0.5 * A)
    half_e0 = jnp.float32(0.5 * A * math.exp(-RC / B))

    dbufs = (dbuf0, dbuf1)
    ibufs = (ibuf0, ibuf1)
    dsems = (sd0, sd1)
    isems = (si0, si1)

    def issue(k, buf):
        eb = (wid + NW * k) * CHUNK
        pltpu.async_copy(dist_hbm.at[pl.ds(eb, CHUNK)], dbufs[buf], dsems[buf])
        pltpu.async_copy(ind_hbm.at[0, pl.ds(eb, CHUNK)], ibufs[buf],
                         isems[buf])

    def wait(buf):
        pltpu.make_async_copy(dist_hbm.at[pl.ds(0, CHUNK)], dbufs[buf],
                              dsems[buf]).wait()
        pltpu.make_async_copy(ind_hbm.at[0, pl.ds(0, CHUNK)], ibufs[buf],
                              isems[buf]).wait()

    def process(buf):
        @plsc.parallel_loop(0, IPC, 1, unroll=U)
        def body(i):
            b16 = i * L
            d = dbufs[buf][pl.ds(b16, L)]
            idxv = ibufs[buf][pl.ds(b16, L)]
            env = half_a * jnp.exp(-d) - half_e0
            row = lax.shift_right_logical(idxv, 7)
            col = lax.bitwise_and(idxv, AC - 1)
            plsc.addupdate_scatter(acc, [row, col], env)

    @pl.when(n_w > 0)
    def _():
        issue(0, 0)

    @pl.when(n_w > 1)
    def _():
        issue(1, 1)

    def zinit(j, carry):
        for k in range(AC // L):
            acc[j, pl.ds(k * L, L)] = zero
        return carry
    lax.fori_loop(0, AR, zinit, 0)

    def iinit(j, carry):
        rowidx[pl.ds(j * L, L)] = io + j * L
        return carry
    lax.fori_loop(0, AR // L, iinit, 0)

    pltpu.sync_copy(acc.at[pl.ds(0, RPT)], stage.at[pl.ds(sid * RPT, RPT)])

    def outer(j, carry):
        k0 = 2 * j
        k1 = 2 * j + 1

        @pl.when(k0 < n_w)
        def _():
            wait(0)
            process(0)

        @pl.when(k0 + 2 < n_w)
        def _():
            issue(k0 + 2, 0)

        @pl.when(k1 < n_w)
        def _():
            wait(1)
            process(1)

        @pl.when(k1 + 2 < n_w)
        def _():
            issue(k1 + 2, 1)

        return carry
    lax.fori_loop(0, (n_w + 1) // 2, outer, 0)

    plsc.subcore_barrier()
    pltpu.sync_copy(acc, stage.at[rowidx], add=True)
    plsc.subcore_barrier()
    pltpu.sync_copy(stage.at[pl.ds(sid * RPT, RPT)],
                    out_hbm.at[cid, pl.ds(sid * RPT, RPT)])


def _tc_add(a_ref, b_ref, o_ref):
    o_ref[...] = a_ref[...] + b_ref[...]


_combine = pl.pallas_call(
    _tc_add,
    out_shape=jax.ShapeDtypeStruct((AR, AC), jnp.float32),
)


def kernel(dist, ind_1, ind_2):
    del ind_1
    idt = ind_2.astype(jnp.int32).T
    partials = _sc_segsum(dist, idt)
    out = _combine(partials[0], partials[1])
    return out.reshape(-1)[:N_ATOMS]


# speedup vs baseline: 98.0437x; 1.3087x over previous
import functools
import math

import jax
import jax.numpy as jnp
from jax import lax
from jax.experimental import pallas as pl
from jax.experimental.pallas import tpu as pltpu
from jax.experimental.pallas import tpu_sc as plsc

RC = 3.0
B = 1.0
A = 1.0
N_ATOMS = 100000
N_EDGES = 6400000

NC = 2
NS = 16
L = 16
NW = NC * NS
CHUNK = 2048
TOTCH = N_EDGES // CHUNK
IPC = CHUNK // L
U = 16
AR = 896
AC = 128
RPT = AR // NS


@functools.partial(
    pl.kernel,
    out_type=jax.ShapeDtypeStruct((NC, AR, AC), jnp.float32),
    mesh=plsc.VectorSubcoreMesh(
        core_axis_name="c", subcore_axis_name="s", num_cores=NC,
        num_subcores=NS,
    ),
    scratch_types=[
        pltpu.VMEM((AR, AC), jnp.float32),
        pltpu.VMEM((CHUNK,), jnp.float32),
        pltpu.VMEM((CHUNK,), jnp.float32),
        pltpu.VMEM((CHUNK,), jnp.int32),
        pltpu.VMEM((CHUNK,), jnp.int32),
        pltpu.VMEM((AR,), jnp.int32),
        pltpu.VMEM_SHARED((AR, AC), jnp.float32),
        pltpu.SemaphoreType.DMA,
        pltpu.SemaphoreType.DMA,
        pltpu.SemaphoreType.DMA,
        pltpu.SemaphoreType.DMA,
    ],
    compiler_params=pltpu.CompilerParams(needs_layout_passes=False),
)
def _sc_segsum(dist_hbm, ind_hbm, out_hbm, acc, dbuf0, dbuf1, ibuf0, ibuf1,
               rowidx, stage, sd0, sd1, si0, si1):
    cid = lax.axis_index("c")
    sid = lax.axis_index("s")
    wid = sid * NC + cid
    n_w = TOTCH // NW + jnp.where(wid < TOTCH % NW, 1, 0)
    io = lax.iota(jnp.int32, L)
    zero = jnp.zeros((L,), jnp.float32)
    half_a = jnp.float32(0.5 * A)
    half_e0 = jnp.float32(0.5 * A * math.exp(-RC / B))

    dbufs = (dbuf0, dbuf1)
    ibufs = (ibuf0, ibuf1)
    dsems = (sd0, sd1)
    isems = (si0, si1)

    def issue(k, buf):
        eb = (wid + NW * k) * CHUNK
        pltpu.async_copy(dist_hbm.at[pl.ds(eb, CHUNK)], dbufs[buf], dsems[buf])
        pltpu.async_copy(ind_hbm.at[0, pl.ds(eb, CHUNK)], ibufs[buf],
                         isems[buf])

    def wait(buf):
        pltpu.make_async_copy(dist_hbm.at[pl.ds(0, CHUNK)], dbufs[buf],
                              dsems[buf]).wait()
        pltpu.make_async_copy(ind_hbm.at[0, pl.ds(0, CHUNK)], ibufs[buf],
                              isems[buf]).wait()

    def process(buf):
        @plsc.parallel_loop(0, IPC, 1, unroll=U)
        def body(i):
            b16 = i * L
            d = dbufs[buf][pl.ds(b16, L)]
            idxv = ibufs[buf][pl.ds(b16, L)]
            env = half_a * jnp.exp(-d) - half_e0
            row = lax.shift_right_logical(idxv, 7)
            col = lax.bitwise_and(idxv, AC - 1)
            plsc.addupdate_scatter(acc, [row, col], env)

    @pl.when(n_w > 0)
    def _():
        issue(0, 0)

    @pl.when(n_w > 1)
    def _():
        issue(1, 1)

    def zinit(j, carry):
        for k in range(AC // L):
            acc[j, pl.ds(k * L, L)] = zero
        return carry
    lax.fori_loop(0, AR, zinit, 0)

    def iinit(j, carry):
        rowidx[pl.ds(j * L, L)] = io + j * L
        return carry
    lax.fori_loop(0, AR // L, iinit, 0)

    pltpu.sync_copy(acc.at[pl.ds(0, RPT)], stage.at[pl.ds(sid * RPT, RPT)])

    def outer(j, carry):
        k0 = 2 * j
        k1 = 2 * j + 1

        @pl.when(k0 < n_w)
        def _():
            wait(0)
            process(0)

        @pl.when(k0 + 2 < n_w)
        def _():
            issue(k0 + 2, 0)

        @pl.when(k1 < n_w)
        def _():
            wait(1)
            process(1)

        @pl.when(k1 + 2 < n_w)
        def _():
            issue(k1 + 2, 1)

        return carry
    lax.fori_loop(0, (n_w + 1) // 2, outer, 0)

    plsc.subcore_barrier()
    pltpu.sync_copy(acc, stage.at[rowidx], add=True)
    plsc.subcore_barrier()
    pltpu.sync_copy(stage.at[pl.ds(sid * RPT, RPT)],
                    out_hbm.at[cid, pl.ds(sid * RPT, RPT)])


def _tc_add(a_ref, b_ref, o_ref):
    o_ref[...] = a_ref[...] + b_ref[...]


_combine = pl.pallas_call(
    _tc_add,
    out_shape=jax.ShapeDtypeStruct((AR, AC), jnp.float32),
)


def kernel(dist, ind_1, ind_2):
    del ind_1
    idt = ind_2.astype(jnp.int32).T
    partials = _sc_segsum(dist, idt)
    out = _combine(partials[0], partials[1])
    return out.reshape(-1)[:N_ATOMS]


# speedup vs baseline: 103.0424x; 1.0510x over previous
import functools
import math

import jax
import jax.numpy as jnp
from jax import lax
from jax.experimental import pallas as pl
from jax.experimental.pallas import tpu as pltpu
from jax.experimental.pallas import tpu_sc as plsc

RC = 3.0
B = 1.0
A = 1.0
N_ATOMS = 100000
N_EDGES = 6400000

NC = 2
NS = 16
L = 16
NW = NC * NS
CHUNK = 2048
TOTCH = N_EDGES // CHUNK
IPC = CHUNK // L
U = 16
AR = 896
AC = 128
RPT = AR // NS


@functools.partial(
    pl.kernel,
    out_type=jax.ShapeDtypeStruct((NC, AR, AC), jnp.float32),
    mesh=plsc.VectorSubcoreMesh(
        core_axis_name="c", subcore_axis_name="s", num_cores=NC,
        num_subcores=NS,
    ),
    scratch_types=[
        pltpu.VMEM((AR, AC), jnp.float32),
        pltpu.VMEM((CHUNK,), jnp.float32),
        pltpu.VMEM((CHUNK,), jnp.float32),
        pltpu.VMEM((CHUNK,), jnp.int32),
        pltpu.VMEM((CHUNK,), jnp.int32),
        pltpu.VMEM((AR,), jnp.int32),
        pltpu.VMEM_SHARED((AR, AC), jnp.float32),
        pltpu.SemaphoreType.DMA,
        pltpu.SemaphoreType.DMA,
        pltpu.SemaphoreType.DMA,
        pltpu.SemaphoreType.DMA,
    ],
    compiler_params=pltpu.CompilerParams(needs_layout_passes=False),
)
def _sc_segsum(dist_hbm, ind_hbm, out_hbm, acc, dbuf0, dbuf1, ibuf0, ibuf1,
               rowidx, stage, sd0, sd1, si0, si1):
    cid = lax.axis_index("c")
    sid = lax.axis_index("s")
    wid = sid * NC + cid
    n_w = TOTCH // NW + jnp.where(wid < TOTCH % NW, 1, 0)
    io = lax.iota(jnp.int32, L)
    zero = jnp.zeros((L,), jnp.float32)
    zcol = jnp.zeros((L,), jnp.int32)
    e0 = jnp.float32(A * math.exp(-RC / B))
    a_const = jnp.float32(A)

    dbufs = (dbuf0, dbuf1)
    ibufs = (ibuf0, ibuf1)
    dsems = (sd0, sd1)
    isems = (si0, si1)

    def issue(k, buf):
        eb = (wid + NW * k) * CHUNK
        pltpu.async_copy(dist_hbm.at[pl.ds(eb, CHUNK)], dbufs[buf], dsems[buf])
        pltpu.async_copy(ind_hbm.at[0, pl.ds(eb, CHUNK)], ibufs[buf],
                         isems[buf])

    def wait(buf):
        pltpu.make_async_copy(dist_hbm.at[pl.ds(0, CHUNK)], dbufs[buf],
                              dsems[buf]).wait()
        pltpu.make_async_copy(ind_hbm.at[0, pl.ds(0, CHUNK)], ibufs[buf],
                              isems[buf]).wait()

    def process(buf):
        @plsc.parallel_loop(0, IPC, 1, unroll=U)
        def body(i):
            b16 = i * L
            d = dbufs[buf][pl.ds(b16, L)]
            idxv = ibufs[buf][pl.ds(b16, L)]
            env = a_const * jnp.exp(-d) - e0
            plsc.addupdate_scatter(acc, [zcol, idxv], env)

    @pl.when(n_w > 0)
    def _():
        issue(0, 0)

    @pl.when(n_w > 1)
    def _():
        issue(1, 1)

    def zinit(j, carry):
        for k in range(AC // L):
            acc[j, pl.ds(k * L, L)] = zero
        return carry
    lax.fori_loop(0, AR, zinit, 0)

    def iinit(j, carry):
        rowidx[pl.ds(j * L, L)] = io + j * L
        return carry
    lax.fori_loop(0, AR // L, iinit, 0)

    pltpu.sync_copy(acc.at[pl.ds(0, RPT)], stage.at[pl.ds(sid * RPT, RPT)])

    def outer(j, carry):
        k0 = 2 * j
        k1 = 2 * j + 1

        @pl.when(k0 < n_w)
        def _():
            wait(0)
            process(0)

        @pl.when(k0 + 2 < n_w)
        def _():
            issue(k0 + 2, 0)

        @pl.when(k1 < n_w)
        def _():
            wait(1)
            process(1)

        @pl.when(k1 + 2 < n_w)
        def _():
            issue(k1 + 2, 1)

        return carry
    lax.fori_loop(0, (n_w + 1) // 2, outer, 0)

    plsc.subcore_barrier()
    pltpu.sync_copy(acc, stage.at[rowidx], add=True)
    plsc.subcore_barrier()
    pltpu.sync_copy(stage.at[pl.ds(sid * RPT, RPT)],
                    out_hbm.at[cid, pl.ds(sid * RPT, RPT)])


def _tc_combine(p_ref, o_ref):
    s = (p_ref[0] + p_ref[1]) * 0.5
    o_ref[...] = s.reshape(-1)[:N_ATOMS]


_combine = pl.pallas_call(
    _tc_combine,
    out_shape=jax.ShapeDtypeStruct((N_ATOMS,), jnp.float32),
)


def kernel(dist, ind_1, ind_2):
    del ind_1
    idt = ind_2.astype(jnp.int32).T
    partials = _sc_segsum(dist, idt)
    return _combine(partials)
